# baseline (speedup 1.0000x reference)
import functools

import numpy as np
import jax
import jax.numpy as jnp
from jax.experimental import pallas as pl
from jax.experimental.pallas import tpu as pltpu

_GRIDS = (1, 2, 4)
_N_PAD = 128
_NB = 2


def _indicator(n: int, g: int) -> np.ndarray:
    m = np.zeros((g, n), dtype=np.float32)
    for i in range(g):
        s = (i * n) // g
        e = -((-(i + 1) * n) // g)
        m[i, s:e] = 1.0
    return m


@functools.lru_cache(maxsize=None)
def _build_operator(h: int, w: int):
    cols, counts = [], []
    for g in sorted(_GRIDS, reverse=True):
        ih, iw = _indicator(h, g), _indicator(w, g)
        cols.append(np.kron(ih, iw).T)
        counts.append(np.outer(ih.sum(1), iw.sum(1)).reshape(-1))
    k = np.concatenate(cols, axis=1)
    n = k.shape[1]
    k_pad = np.zeros((h * w, _N_PAD), dtype=np.float32)
    k_pad[:, :n] = k
    s_pad = np.zeros((1, _N_PAD), dtype=np.float32)
    s_pad[0, :n] = 1.0 / np.concatenate(counts)
    return k_pad, s_pad


def _spp_kernel(x_ref, k_ref, s_ref, o4_ref, o2_ref, o1_ref):
    xb = x_ref[...].astype(jnp.bfloat16)
    p = jnp.dot(xb, k_ref[...], preferred_element_type=jnp.float32)
    p = p * s_ref[...]
    o4_ref[...] = p[:, 0:16].reshape(o4_ref.shape)
    o2_ref[...] = p[:, 16:20].reshape(o2_ref.shape)
    o1_ref[...] = p[:, 20:21].reshape(o1_ref.shape)


def kernel(x):
    B, C, H, W = x.shape
    HW = H * W
    x2d = x.reshape(B * C, HW)

    k_np, s_np = _build_operator(H, W)
    k_op = jnp.asarray(k_np, dtype=jnp.bfloat16)
    s_op = jnp.asarray(s_np, dtype=jnp.float32)

    nb = _NB if B % _NB == 0 else 1
    grid = (B // nb,)
    dt = x.dtype

    o4, o2, o1 = pl.pallas_call(
        _spp_kernel,
        out_shape=[
            jax.ShapeDtypeStruct((B, C, 16), dt),
            jax.ShapeDtypeStruct((B, C, 4), dt),
            jax.ShapeDtypeStruct((B, C, 1), dt),
        ],
        grid=grid,
        in_specs=[
            pl.BlockSpec((nb * C, HW), lambda i: (i, 0)),
            pl.BlockSpec((HW, _N_PAD), lambda i: (0, 0)),
            pl.BlockSpec((1, _N_PAD), lambda i: (0, 0)),
        ],
        out_specs=[
            pl.BlockSpec((nb, C, 16), lambda i: (i, 0, 0)),
            pl.BlockSpec((nb, C, 4), lambda i: (i, 0, 0)),
            pl.BlockSpec((nb, C, 1), lambda i: (i, 0, 0)),
        ],
        compiler_params=pltpu.CompilerParams(
            dimension_semantics=("parallel",),
            vmem_limit_bytes=64 * 1024 * 1024,
        ),
    )(x2d, k_op, s_op)

    out = jnp.concatenate(
        [o1.reshape(B, C), o2.reshape(B, 4 * C), o4.reshape(B, 16 * C)],
        axis=1,
    )
    return out.reshape(B, 21 * C, 1, 1)


# speedup vs baseline: 1.0430x; 1.0430x over previous
import functools

import numpy as np
import jax
import jax.numpy as jnp
from jax.experimental import pallas as pl
from jax.experimental.pallas import tpu as pltpu

_GRIDS = (1, 2, 4)
_N_PAD = 128
_NB = 1


def _indicator(n: int, g: int) -> np.ndarray:
    m = np.zeros((g, n), dtype=np.float32)
    for i in range(g):
        s = (i * n) // g
        e = -((-(i + 1) * n) // g)
        m[i, s:e] = 1.0
    return m


@functools.lru_cache(maxsize=None)
def _build_operator(h: int, w: int):
    cols, counts = [], []
    for g in sorted(_GRIDS, reverse=True):
        ih, iw = _indicator(h, g), _indicator(w, g)
        cols.append(np.kron(ih, iw).T)
        counts.append(np.outer(ih.sum(1), iw.sum(1)).reshape(-1))
    k = np.concatenate(cols, axis=1)
    n = k.shape[1]
    k_pad = np.zeros((h * w, _N_PAD), dtype=np.float32)
    k_pad[:, :n] = k
    s_pad = np.zeros((1, _N_PAD), dtype=np.float32)
    s_pad[0, :n] = 1.0 / np.concatenate(counts)
    return k_pad, s_pad


def _spp_kernel(x_ref, k_ref, s_ref, o_ref):
    rows = x_ref.shape[0]
    hw = x_ref.shape[1] * x_ref.shape[2]
    nb = o_ref.shape[0]
    c = rows // nb
    r1, r2 = c // 128, c // 32
    xb = x_ref[...].astype(jnp.bfloat16).reshape(rows, hw)
    p = jnp.dot(xb, k_ref[...], preferred_element_type=jnp.float32)
    p = p * s_ref[...]
    for i in range(nb):
        pi = p[i * c:(i + 1) * c]
        s1 = pi.reshape(r1, 128, 128)[:, :, 20:21].reshape(r1, 128)
        s2 = pi.reshape(r2, 32, 128)[:, :, 16:20].reshape(r2, 128)
        s4 = pi.reshape(c // 8, 8, 128)[:, :, 0:16].reshape(c // 8, 128)
        o_ref[i, 0:r1, :] = s1
        o_ref[i, r1:r1 + r2, :] = s2
        o_ref[i, r1 + r2:, :] = s4


def kernel(x):
    B, C, H, W = x.shape
    HW = H * W
    x3 = x.reshape(B * C, H, W)

    k_np, s_np = _build_operator(H, W)
    k_op = jnp.asarray(k_np, dtype=jnp.bfloat16)
    s_op = jnp.asarray(s_np, dtype=jnp.float32)

    nb = _NB if B % _NB == 0 else 1
    grid = (B // nb,)
    dt = x.dtype

    out_rows = (21 * C) // 128
    out = pl.pallas_call(
        _spp_kernel,
        out_shape=jax.ShapeDtypeStruct((B, out_rows, 128), dt),
        grid=grid,
        in_specs=[
            pl.BlockSpec((nb * C, H, W), lambda i: (i, 0, 0)),
            pl.BlockSpec((HW, _N_PAD), lambda i: (0, 0)),
            pl.BlockSpec((1, _N_PAD), lambda i: (0, 0)),
        ],
        out_specs=pl.BlockSpec((nb, out_rows, 128), lambda i: (i, 0, 0)),
        compiler_params=pltpu.CompilerParams(
            dimension_semantics=("parallel",),
            vmem_limit_bytes=64 * 1024 * 1024,
        ),
    )(x3, k_op, s_op)

    return out.reshape(B, 21 * C, 1, 1)


# speedup vs baseline: 5.7489x; 5.5120x over previous
import functools

import numpy as np
import jax
import jax.numpy as jnp
from jax.experimental import pallas as pl
from jax.experimental.pallas import tpu as pltpu

_CQ = 128


def _window_starts(n: int, g: int):
    return [((i * n) // g, -((-(i + 1) * n) // g)) for i in range(g)]


@functools.lru_cache(maxsize=None)
def _perm_matrix(gg: int, cq: int):
    g = np.zeros((gg * cq, gg * cq), dtype=np.float32)
    for k in range(gg):
        for m in range(cq):
            g[cq * k + m, gg * m + k] = 1.0
    return g


def _spp_kernel(x_ref, g4_ref, g2_ref, o1_ref, o2_ref, o4_ref, *, h, w):
    hs4 = _window_starts(h, 4)
    ws4 = _window_starts(w, 4)
    inv_hw = 1.0 / float(h * w)
    inv2 = 4.0 * inv_hw
    inv4 = 16.0 * inv_hw

    s4 = []
    for i in range(4):
        for j in range(4):
            acc = None
            for a in range(hs4[i][0], hs4[i][1]):
                for b in range(ws4[j][0], ws4[j][1]):
                    t = x_ref[a * w + b]
                    acc = t if acc is None else acc + t
            s4.append(acc)
    z2 = [s4[4 * (2 * i2) + 2 * j2] + s4[4 * (2 * i2) + 2 * j2 + 1]
          + s4[4 * (2 * i2 + 1) + 2 * j2] + s4[4 * (2 * i2 + 1) + 2 * j2 + 1]
          for i2 in range(2) for j2 in range(2)]
    z1 = z2[0] + z2[1] + z2[2] + z2[3]

    o1_ref[...] = z1 * inv_hw
    zc2 = jnp.concatenate(z2, axis=1).astype(jnp.bfloat16)
    o2_ref[...] = jnp.dot(
        zc2, g2_ref[...], preferred_element_type=jnp.float32) * inv2
    zc4 = jnp.concatenate(s4, axis=1).astype(jnp.bfloat16)
    o4_ref[...] = jnp.dot(
        zc4, g4_ref[...], preferred_element_type=jnp.float32) * inv4


def kernel(x):
    B, C, H, W = x.shape
    HW = H * W
    xt = jnp.transpose(x, (2, 3, 0, 1)).reshape(HW, B, C)

    g4_op = jnp.asarray(_perm_matrix(16, _CQ), dtype=jnp.bfloat16)
    g2_op = jnp.asarray(_perm_matrix(4, _CQ), dtype=jnp.bfloat16)

    nq = C // _CQ
    dt = x.dtype

    o1, o2, o4 = pl.pallas_call(
        functools.partial(_spp_kernel, h=H, w=W),
        out_shape=[
            jax.ShapeDtypeStruct((B, C), dt),
            jax.ShapeDtypeStruct((B, 4 * C), dt),
            jax.ShapeDtypeStruct((B, 16 * C), dt),
        ],
        grid=(nq,),
        in_specs=[
            pl.BlockSpec((HW, B, _CQ), lambda q: (0, 0, q)),
            pl.BlockSpec((16 * _CQ, 16 * _CQ), lambda q: (0, 0)),
            pl.BlockSpec((4 * _CQ, 4 * _CQ), lambda q: (0, 0)),
        ],
        out_specs=[
            pl.BlockSpec((B, _CQ), lambda q: (0, q)),
            pl.BlockSpec((B, 4 * _CQ), lambda q: (0, q)),
            pl.BlockSpec((B, 16 * _CQ), lambda q: (0, q)),
        ],
        compiler_params=pltpu.CompilerParams(
            dimension_semantics=("parallel",),
            vmem_limit_bytes=64 * 1024 * 1024,
        ),
    )(xt, g4_op, g2_op)

    out = jnp.concatenate([o1, o2, o4], axis=1)
    return out.reshape(B, 21 * C, 1, 1)
